# baseline (speedup 1.0000x reference)
import math

import jax
import jax.numpy as jnp
from jax.experimental import pallas as pl
from jax.experimental.pallas import tpu as pltpu

N_CLUSTER = 1024
N_FEATURES = 256
BLOCK_B = 1024


def _gmm_kernel(z_ref, lsT_ref, muT_ref, pi_ref, yc_ref, oh_ref,
                at_ref, bt_ref, r_ref):
    i = pl.program_id(0)

    @pl.when(i == 0)
    def _prologue():
        lsT = lsT_ref[...]
        muT = muT_ref[...]
        inv_sT = jnp.exp(-lsT)
        log2e = 1.4426950408889634
        at_ref[...] = (-0.5 * log2e) * inv_sT
        bt_ref[...] = log2e * (muT * inv_sT)
        const = jnp.sum(lsT, axis=0, keepdims=True)
        c = jnp.sum(muT * muT * inv_sT, axis=0, keepdims=True)
        logpi = jnp.log(pi_ref[...])
        r_ref[...] = log2e * (logpi - 0.5 * (const + c
                                    + N_FEATURES * math.log(2.0 * math.pi)))

    z = z_ref[...]
    zz = z * z
    logits = (r_ref[...]
              + jnp.dot(zz, at_ref[...], preferred_element_type=jnp.float32)
              + jnp.dot(z, bt_ref[...], preferred_element_type=jnp.float32))
    yita = jnp.exp2(logits) + 1e-10
    s = jnp.sum(yita, axis=1, keepdims=True)
    yc = yita * (1.0 / s)
    yc_ref[...] = yc

    idx = jnp.argmax(yc, axis=1)[:, None].astype(jnp.int32)
    iota = jax.lax.broadcasted_iota(jnp.int32, yc.shape, 1)
    oh_ref[...] = (iota == idx).astype(jnp.float32)


@jax.jit
def kernel(z, pi_, mu_c, log_sigma2_c):
    B, d = z.shape
    K = mu_c.shape[0]
    grid = (B // BLOCK_B,)
    lsT = log_sigma2_c.T
    muT = mu_c.T
    pi2 = pi_.reshape(1, K)

    yc, oh = pl.pallas_call(
        _gmm_kernel,
        grid=grid,
        in_specs=[
            pl.BlockSpec((BLOCK_B, d), lambda i: (i, 0)),
            pl.BlockSpec((d, K), lambda i: (0, 0)),
            pl.BlockSpec((d, K), lambda i: (0, 0)),
            pl.BlockSpec((1, K), lambda i: (0, 0)),
        ],
        out_specs=[
            pl.BlockSpec((BLOCK_B, K), lambda i: (i, 0)),
            pl.BlockSpec((BLOCK_B, K), lambda i: (i, 0)),
        ],
        out_shape=[
            jax.ShapeDtypeStruct((B, K), jnp.float32),
            jax.ShapeDtypeStruct((B, K), jnp.float32),
        ],
        scratch_shapes=[
            pltpu.VMEM((d, K), jnp.float32),
            pltpu.VMEM((d, K), jnp.float32),
            pltpu.VMEM((1, K), jnp.float32),
        ],
        compiler_params=pltpu.CompilerParams(
            dimension_semantics=("arbitrary",),
        ),
    )(z, lsT, muT, pi2)
    return (yc, oh)


# speedup vs baseline: 1.0983x; 1.0983x over previous
import math

import jax
import jax.numpy as jnp
from jax.experimental import pallas as pl
from jax.experimental.pallas import tpu as pltpu

N_CLUSTER = 1024
N_FEATURES = 256
BLOCK_B = 1024


def _gmm_kernel(z_ref, lsT_ref, muT_ref, pi_ref, yc_ref, oh_ref,
                at_ref, bt_ref, r_ref):
    i = pl.program_id(0)

    @pl.when(i == 0)
    def _prologue():
        lsT = lsT_ref[...]
        muT = muT_ref[...]
        inv_sT = jnp.exp(-lsT)
        log2e = 1.4426950408889634
        at_ref[...] = (-0.5 * log2e) * inv_sT
        bt_ref[...] = log2e * (muT * inv_sT)
        const = jnp.sum(lsT, axis=0, keepdims=True)
        c = jnp.sum(muT * muT * inv_sT, axis=0, keepdims=True)
        logpi = jnp.log(pi_ref[...])
        r_ref[...] = log2e * (logpi - 0.5 * (const + c
                                    + N_FEATURES * math.log(2.0 * math.pi)))

    z = z_ref[...]
    zz = z * z
    logits = (r_ref[...]
              + jnp.dot(zz, at_ref[...], preferred_element_type=jnp.float32)
              + jnp.dot(z, bt_ref[...], preferred_element_type=jnp.float32))
    yita = jnp.exp2(logits) + 1e-10
    s = jnp.sum(yita, axis=1, keepdims=True)
    yc = yita * (1.0 / s)
    yc_ref[...] = yc

    m = jnp.max(yc, axis=1, keepdims=True)
    iota = jax.lax.broadcasted_iota(jnp.int32, yc.shape, 1)
    idx = jnp.min(jnp.where(yc == m, iota, N_CLUSTER), axis=1, keepdims=True)
    oh_ref[...] = (iota == idx).astype(jnp.float32)


@jax.jit
def kernel(z, pi_, mu_c, log_sigma2_c):
    B, d = z.shape
    K = mu_c.shape[0]
    grid = (B // BLOCK_B,)
    lsT = log_sigma2_c.T
    muT = mu_c.T
    pi2 = pi_.reshape(1, K)

    yc, oh = pl.pallas_call(
        _gmm_kernel,
        grid=grid,
        in_specs=[
            pl.BlockSpec((BLOCK_B, d), lambda i: (i, 0)),
            pl.BlockSpec((d, K), lambda i: (0, 0)),
            pl.BlockSpec((d, K), lambda i: (0, 0)),
            pl.BlockSpec((1, K), lambda i: (0, 0)),
        ],
        out_specs=[
            pl.BlockSpec((BLOCK_B, K), lambda i: (i, 0)),
            pl.BlockSpec((BLOCK_B, K), lambda i: (i, 0)),
        ],
        out_shape=[
            jax.ShapeDtypeStruct((B, K), jnp.float32),
            jax.ShapeDtypeStruct((B, K), jnp.float32),
        ],
        scratch_shapes=[
            pltpu.VMEM((d, K), jnp.float32),
            pltpu.VMEM((d, K), jnp.float32),
            pltpu.VMEM((1, K), jnp.float32),
        ],
        compiler_params=pltpu.CompilerParams(
            dimension_semantics=("arbitrary",),
        ),
    )(z, lsT, muT, pi2)
    return (yc, oh)
